# Initial kernel scaffold; baseline (speedup 1.0000x reference)
import jax
import jax.numpy as jnp
from jax.experimental import pallas as pl


def kernel(x, adj, weight, bias):
    raise NotImplementedError("write your pallas kernel here")


# speedup vs baseline: 2.6173x; 2.6173x over previous
import functools

import jax
import jax.numpy as jnp
from jax.experimental import pallas as pl
from jax.experimental.pallas import tpu as pltpu

B, N, CIN, COUT, K = 2, 2048, 128, 128, 3
BLK = 256


def _deg_cast_body(adj_ref, d_ref, adjb_ref):
    a = adj_ref[...]
    d_ref[...] = jax.lax.rsqrt(jnp.sum(a, axis=1, keepdims=True))
    adjb_ref[...] = a.astype(jnp.bfloat16)


def _deg_cast(adj):
    return pl.pallas_call(
        _deg_cast_body,
        grid=(N // BLK,),
        in_specs=[pl.BlockSpec((BLK, N), lambda i: (i, 0))],
        out_specs=(
            pl.BlockSpec((BLK, 1), lambda i: (i, 0)),
            pl.BlockSpec((BLK, N), lambda i: (i, 0)),
        ),
        out_shape=(
            jax.ShapeDtypeStruct((N, 1), jnp.float32),
            jax.ShapeDtypeStruct((N, N), jnp.bfloat16),
        ),
    )(adj)


def _prop_body(adjb_ref, d_ref, xs_ref, p_ref, u_ref, *, subtract_ref=None):
    i = pl.program_id(0)

    @pl.when(i == 0)
    def _():
        z = xs_ref[...]
        if subtract_ref is not None:
            z = z - subtract_ref[...]
        u_ref[...] = (z * d_ref[...]).astype(jnp.bfloat16)

    acc = jnp.dot(adjb_ref[...], u_ref[...], preferred_element_type=jnp.float32)
    p_ref[...] = acc * d_ref[pl.ds(i * BLK, BLK), :]


def _prop1_body(adjb_ref, d_ref, xs_ref, p_ref, u_ref):
    _prop_body(adjb_ref, d_ref, xs_ref, p_ref, u_ref)


def _prop2_body(adjb_ref, d_ref, xs_ref, pprev_ref, q_ref, u_ref):
    def inner(adjb, d, xs, q, u):
        _prop_body(adjb, d, xs, q, u, subtract_ref=pprev_ref)
    inner(adjb_ref, d_ref, xs_ref, q_ref, u_ref)


def _prop(adjb, d, xs, pprev=None):
    full2 = lambda shape: pl.BlockSpec(shape, lambda i: (0, 0))
    in_specs = [
        pl.BlockSpec((BLK, N), lambda i: (i, 0)),
        full2((N, 1)),
        full2((N, B * CIN)),
    ]
    args = [adjb, d, xs]
    body = _prop1_body
    if pprev is not None:
        in_specs.append(full2((N, B * CIN)))
        args.append(pprev)
        body = _prop2_body
    return pl.pallas_call(
        body,
        grid=(N // BLK,),
        in_specs=in_specs,
        out_specs=pl.BlockSpec((BLK, B * CIN), lambda i: (i, 0)),
        out_shape=jax.ShapeDtypeStruct((N, B * CIN), jnp.float32),
        scratch_shapes=[pltpu.VMEM((N, B * CIN), jnp.bfloat16)],
    )(*args)


def _combine_body(xs_ref, p_ref, q_ref, w_ref, b_ref, out_ref):
    w0 = w_ref[0, 0]
    w1 = w_ref[1, 0]
    w2 = w_ref[2, 0]
    wa = w0 + w1 + w2
    wb = -(w1 + 2.0 * w2)
    wc = -2.0 * w2
    bias = b_ref[0, 0, :]
    for b in range(B):
        sl = slice(b * CIN, (b + 1) * CIN)
        r = jnp.dot(xs_ref[:, sl], wa, preferred_element_type=jnp.float32)
        r = r + jnp.dot(p_ref[:, sl], wb, preferred_element_type=jnp.float32)
        r = r + jnp.dot(q_ref[:, sl], wc, preferred_element_type=jnp.float32)
        out_ref[b] = jnp.maximum(r + bias, 0.0)


def _combine(xs, p, q, weight, bias):
    return pl.pallas_call(
        _combine_body,
        out_shape=jax.ShapeDtypeStruct((B, N, COUT), jnp.float32),
    )(xs, p, q, weight, bias)


@jax.jit
def kernel(x, adj, weight, bias):
    xs = jnp.transpose(x, (1, 0, 2)).reshape(N, B * CIN)
    d, adjb = _deg_cast(adj)
    p = _prop(adjb, d, xs)
    q = _prop(adjb, d, xs, pprev=p)
    return _combine(xs, p, q, weight, bias)


# speedup vs baseline: 4.2094x; 1.6083x over previous
import jax
import jax.numpy as jnp
from jax.experimental import pallas as pl
from jax.experimental.pallas import tpu as pltpu

B, N, CIN, COUT = 2, 2048, 128, 128
BLK = 256
NBLK = N // BLK


def _fused_body(adj_ref, xs_ref, w_ref, b_ref, out_ref,
                adjb_s, d_s, ps_s, u_s):
    s = pl.program_id(0)

    @pl.when(s < NBLK)
    def _():
        a = adj_ref[...]
        rows = pl.ds(s * BLK, BLK)
        d_s[rows, :] = jax.lax.rsqrt(jnp.sum(a, axis=1, keepdims=True))
        adjb_s[rows, :] = a.astype(jnp.bfloat16)

    @pl.when(s == NBLK)
    def _():
        u_s[...] = (xs_ref[...] * d_s[...]).astype(jnp.bfloat16)

    @pl.when((s >= NBLK) & (s < 2 * NBLK))
    def _():
        rows = pl.ds((s - NBLK) * BLK, BLK)
        acc = jnp.dot(adjb_s[rows, :], u_s[...],
                      preferred_element_type=jnp.float32)
        ps_s[rows, :] = acc * d_s[rows, :]

    @pl.when(s == 2 * NBLK)
    def _():
        u_s[...] = ((xs_ref[...] - ps_s[...]) * d_s[...]).astype(jnp.bfloat16)

    @pl.when(s >= 2 * NBLK)
    def _():
        rows = pl.ds((s - 2 * NBLK) * BLK, BLK)
        acc = jnp.dot(adjb_s[rows, :], u_s[...],
                      preferred_element_type=jnp.float32)
        q = acc * d_s[rows, :]
        p = ps_s[rows, :]
        xb = xs_ref[rows, :]
        w0 = w_ref[0, 0]
        w1 = w_ref[1, 0]
        w2 = w_ref[2, 0]
        wa = w0 + w1 + w2
        wb = -(w1 + 2.0 * w2)
        wc = -2.0 * w2
        bias = b_ref[0, 0, :]
        for b in range(B):
            sl = slice(b * CIN, (b + 1) * CIN)
            r = jnp.dot(xb[:, sl], wa, preferred_element_type=jnp.float32)
            r = r + jnp.dot(p[:, sl], wb, preferred_element_type=jnp.float32)
            r = r + jnp.dot(q[:, sl], wc, preferred_element_type=jnp.float32)
            out_ref[b] = jnp.maximum(r + bias, 0.0)


@jax.jit
def kernel(x, adj, weight, bias):
    xs = jnp.transpose(x, (1, 0, 2)).reshape(N, B * CIN)
    return pl.pallas_call(
        _fused_body,
        grid=(3 * NBLK,),
        in_specs=[
            pl.BlockSpec((BLK, N), lambda s: (jnp.minimum(s, NBLK - 1), 0)),
            pl.BlockSpec((N, B * CIN), lambda s: (0, 0)),
            pl.BlockSpec((3, 1, CIN, COUT), lambda s: (0, 0, 0, 0)),
            pl.BlockSpec((1, 1, COUT), lambda s: (0, 0, 0)),
        ],
        out_specs=pl.BlockSpec(
            (B, BLK, COUT), lambda s: (0, jnp.maximum(s - 2 * NBLK, 0), 0)),
        out_shape=jax.ShapeDtypeStruct((B, N, COUT), jnp.float32),
        scratch_shapes=[
            pltpu.VMEM((N, N), jnp.bfloat16),
            pltpu.VMEM((N, 1), jnp.float32),
            pltpu.VMEM((N, B * CIN), jnp.float32),
            pltpu.VMEM((N, B * CIN), jnp.bfloat16),
        ],
    )(adj, xs, weight, bias)


# speedup vs baseline: 4.9536x; 1.1768x over previous
import jax
import jax.numpy as jnp
from jax.experimental import pallas as pl
from jax.experimental.pallas import tpu as pltpu

B, N, CIN, COUT = 2, 2048, 128, 128
BLK = 256
NBLK = N // BLK
PROP_DTYPE = jnp.float8_e4m3fn


def _fused_body(adj_ref, xs_ref, w_ref, b_ref, out_ref,
                adjb_s, d_s, ps_s, u_s):
    s = pl.program_id(0)

    @pl.when(s < NBLK)
    def _():
        a = adj_ref[...]
        rows = pl.ds(s * BLK, BLK)
        d_s[rows, :] = jax.lax.rsqrt(jnp.sum(a, axis=1, keepdims=True))
        adjb_s[rows, :] = a.astype(PROP_DTYPE)

    @pl.when(s == NBLK)
    def _():
        u_s[...] = (xs_ref[...] * d_s[...]).astype(PROP_DTYPE)

    @pl.when((s >= NBLK) & (s < 2 * NBLK))
    def _():
        rows = pl.ds((s - NBLK) * BLK, BLK)
        acc = jnp.dot(adjb_s[rows, :], u_s[...],
                      preferred_element_type=jnp.float32)
        ps_s[rows, :] = acc * d_s[rows, :]

    @pl.when(s == 2 * NBLK)
    def _():
        u_s[...] = ((xs_ref[...] - ps_s[...]) * d_s[...]).astype(PROP_DTYPE)

    @pl.when(s >= 2 * NBLK)
    def _():
        rows = pl.ds((s - 2 * NBLK) * BLK, BLK)
        acc = jnp.dot(adjb_s[rows, :], u_s[...],
                      preferred_element_type=jnp.float32)
        q = acc * d_s[rows, :]
        p = ps_s[rows, :]
        xb = xs_ref[rows, :]
        w0 = w_ref[0, 0]
        w1 = w_ref[1, 0]
        w2 = w_ref[2, 0]
        wa = w0 + w1 + w2
        wb = -(w1 + 2.0 * w2)
        wc = -2.0 * w2
        bias = b_ref[0, 0, :]
        for b in range(B):
            sl = slice(b * CIN, (b + 1) * CIN)
            r = jnp.dot(xb[:, sl], wa, preferred_element_type=jnp.float32)
            r = r + jnp.dot(p[:, sl], wb, preferred_element_type=jnp.float32)
            r = r + jnp.dot(q[:, sl], wc, preferred_element_type=jnp.float32)
            out_ref[b] = jnp.maximum(r + bias, 0.0)


@jax.jit
def kernel(x, adj, weight, bias):
    xs = jnp.transpose(x, (1, 0, 2)).reshape(N, B * CIN)
    return pl.pallas_call(
        _fused_body,
        grid=(3 * NBLK,),
        in_specs=[
            pl.BlockSpec((BLK, N), lambda s: (jnp.minimum(s, NBLK - 1), 0)),
            pl.BlockSpec((N, B * CIN), lambda s: (0, 0)),
            pl.BlockSpec((3, 1, CIN, COUT), lambda s: (0, 0, 0, 0)),
            pl.BlockSpec((1, 1, COUT), lambda s: (0, 0, 0)),
        ],
        out_specs=pl.BlockSpec(
            (B, BLK, COUT), lambda s: (0, jnp.maximum(s - 2 * NBLK, 0), 0)),
        out_shape=jax.ShapeDtypeStruct((B, N, COUT), jnp.float32),
        scratch_shapes=[
            pltpu.VMEM((N, N), PROP_DTYPE),
            pltpu.VMEM((N, 1), jnp.float32),
            pltpu.VMEM((N, B * CIN), jnp.float32),
            pltpu.VMEM((N, B * CIN), PROP_DTYPE),
        ],
    )(adj, xs, weight, bias)


# speedup vs baseline: 5.2779x; 1.0655x over previous
import jax
import jax.numpy as jnp
from jax.experimental import pallas as pl
from jax.experimental.pallas import tpu as pltpu

B, N, CIN, COUT = 2, 2048, 128, 128
BLK = 256
NBLK = N // BLK
PROP_DTYPE = jnp.float8_e4m3fn
Q0 = NBLK + 1


def _fused_body(adj_ref, xs_ref, w_ref, b_ref, out_ref,
                adjb_s, d_s, ps_s, u_s):
    s = pl.program_id(0)

    @pl.when(s < NBLK)
    def _():
        a = adj_ref[...]
        rows = pl.ds(s * BLK, BLK)
        d = jax.lax.rsqrt(jnp.sum(a, axis=1, keepdims=True))
        d_s[rows, :] = d
        adjb_s[rows, :] = a.astype(PROP_DTYPE)
        u_s[rows, :] = (xs_ref[rows, :] * d).astype(PROP_DTYPE)

    for t in range(NBLK):
        @pl.when(s == t + 1)
        def _(t=t):
            hi = (t + 1) * BLK
            ps_s[t * BLK:hi, :] = jnp.dot(
                adjb_s[t * BLK:hi, :hi], u_s[:hi, :],
                preferred_element_type=jnp.float32)
            if t > 0:
                ps_s[:t * BLK, :] += jnp.dot(
                    adjb_s[:t * BLK, t * BLK:hi], u_s[t * BLK:hi, :],
                    preferred_element_type=jnp.float32)

    @pl.when(s == Q0)
    def _():
        p = ps_s[...] * d_s[...]
        ps_s[...] = p
        u_s[...] = ((xs_ref[...] - p) * d_s[...]).astype(PROP_DTYPE)

    @pl.when(s >= Q0)
    def _():
        rows = pl.ds((s - Q0) * BLK, BLK)
        acc = jnp.dot(adjb_s[rows, :], u_s[...],
                      preferred_element_type=jnp.float32)
        q = acc * d_s[rows, :]
        p = ps_s[rows, :]
        xb = xs_ref[rows, :]
        w0 = w_ref[0, 0]
        w1 = w_ref[1, 0]
        w2 = w_ref[2, 0]
        wa = w0 + w1 + w2
        wb = -(w1 + 2.0 * w2)
        wc = -2.0 * w2
        bias = b_ref[0, 0, :]
        for b in range(B):
            sl = slice(b * CIN, (b + 1) * CIN)
            r = jnp.dot(xb[:, sl], wa, preferred_element_type=jnp.float32)
            r = r + jnp.dot(p[:, sl], wb, preferred_element_type=jnp.float32)
            r = r + jnp.dot(q[:, sl], wc, preferred_element_type=jnp.float32)
            out_ref[b] = jnp.maximum(r + bias, 0.0)


@jax.jit
def kernel(x, adj, weight, bias):
    xs = jnp.transpose(x, (1, 0, 2)).reshape(N, B * CIN)
    return pl.pallas_call(
        _fused_body,
        grid=(Q0 + NBLK,),
        in_specs=[
            pl.BlockSpec((BLK, N), lambda s: (jnp.minimum(s, NBLK - 1), 0)),
            pl.BlockSpec((N, B * CIN), lambda s: (0, 0)),
            pl.BlockSpec((3, 1, CIN, COUT), lambda s: (0, 0, 0, 0)),
            pl.BlockSpec((1, 1, COUT), lambda s: (0, 0, 0)),
        ],
        out_specs=pl.BlockSpec(
            (B, BLK, COUT), lambda s: (0, jnp.maximum(s - Q0, 0), 0)),
        out_shape=jax.ShapeDtypeStruct((B, N, COUT), jnp.float32),
        scratch_shapes=[
            pltpu.VMEM((N, N), PROP_DTYPE),
            pltpu.VMEM((N, 1), jnp.float32),
            pltpu.VMEM((N, B * CIN), jnp.float32),
            pltpu.VMEM((N, B * CIN), PROP_DTYPE),
        ],
    )(adj, xs, weight, bias)


# speedup vs baseline: 5.8770x; 1.1135x over previous
import jax
import jax.numpy as jnp
from jax.experimental import pallas as pl
from jax.experimental.pallas import tpu as pltpu

B, N, CIN, COUT = 2, 2048, 128, 128
BS = 512
NS = N // BS
BQ = 256
NQ = N // BQ
Q0 = NS + 1
PROP_DTYPE = jnp.float8_e4m3fn


def _fused_body(adj_ref, xs_ref, w_ref, b_ref, out_ref,
                adjb_s, d_s, ps_s, u_s):
    s = pl.program_id(0)

    @pl.when(s < NS)
    def _():
        rows = pl.ds(s * BS, BS)
        af8 = adj_ref[...].astype(PROP_DTYPE)
        adjb_s[rows, :] = af8
        ones = jnp.ones((N, 128), PROP_DTYPE)
        deg = jnp.dot(af8, ones, preferred_element_type=jnp.float32)
        d = jax.lax.rsqrt(deg[:, :1])
        d_s[rows, :] = d
        u_s[rows, :] = (xs_ref[rows, :] * d).astype(PROP_DTYPE)

    for t in range(NS):
        @pl.when(s == t + 1)
        def _(t=t):
            hi = (t + 1) * BS
            ps_s[t * BS:hi, :] = jnp.dot(
                adjb_s[t * BS:hi, :hi], u_s[:hi, :],
                preferred_element_type=jnp.float32)
            if t > 0:
                ps_s[:t * BS, :] += jnp.dot(
                    adjb_s[:t * BS, t * BS:hi], u_s[t * BS:hi, :],
                    preferred_element_type=jnp.float32)

    @pl.when(s == NS)
    def _():
        p = ps_s[...] * d_s[...]
        ps_s[...] = p
        u_s[...] = ((xs_ref[...] - p) * d_s[...]).astype(PROP_DTYPE)

    @pl.when(s >= Q0)
    def _():
        rows = pl.ds((s - Q0) * BQ, BQ)
        acc = jnp.dot(adjb_s[rows, :], u_s[...],
                      preferred_element_type=jnp.float32)
        q = (acc * d_s[rows, :]).astype(jnp.bfloat16)
        p = ps_s[rows, :].astype(jnp.bfloat16)
        xb = xs_ref[rows, :]
        w0 = w_ref[0, 0]
        w1 = w_ref[1, 0]
        w2 = w_ref[2, 0]
        wa = w0 + w1 + w2
        wb = (-(w1 + 2.0 * w2)).astype(jnp.bfloat16)
        wc = (-2.0 * w2).astype(jnp.bfloat16)
        bias = b_ref[0, 0, :]
        for b in range(B):
            sl = slice(b * CIN, (b + 1) * CIN)
            r = jnp.dot(xb[:, sl], wa, preferred_element_type=jnp.float32)
            r = r + jnp.dot(p[:, sl], wb, preferred_element_type=jnp.float32)
            r = r + jnp.dot(q[:, sl], wc, preferred_element_type=jnp.float32)
            out_ref[b] = jnp.maximum(r + bias, 0.0)


@jax.jit
def kernel(x, adj, weight, bias):
    xs = jnp.transpose(x, (1, 0, 2)).reshape(N, B * CIN)
    return pl.pallas_call(
        _fused_body,
        grid=(Q0 + NQ,),
        in_specs=[
            pl.BlockSpec((BS, N), lambda s: (jnp.minimum(s, NS - 1), 0)),
            pl.BlockSpec((N, B * CIN), lambda s: (0, 0)),
            pl.BlockSpec((3, 1, CIN, COUT), lambda s: (0, 0, 0, 0)),
            pl.BlockSpec((1, 1, COUT), lambda s: (0, 0, 0)),
        ],
        out_specs=pl.BlockSpec(
            (B, BQ, COUT), lambda s: (0, jnp.maximum(s - Q0, 0), 0)),
        out_shape=jax.ShapeDtypeStruct((B, N, COUT), jnp.float32),
        scratch_shapes=[
            pltpu.VMEM((N, N), PROP_DTYPE),
            pltpu.VMEM((N, 1), jnp.float32),
            pltpu.VMEM((N, B * CIN), jnp.float32),
            pltpu.VMEM((N, B * CIN), PROP_DTYPE),
        ],
    )(adj, xs, weight, bias)


# speedup vs baseline: 6.3832x; 1.0861x over previous
import jax
import jax.numpy as jnp
from jax.experimental import pallas as pl
from jax.experimental.pallas import tpu as pltpu

B, N, CIN, COUT = 2, 2048, 128, 128
BS = 512
NS = N // BS
BQ = 256
NQ = N // BQ
Q0 = NS + 1
PROP_DTYPE = jnp.float8_e4m3fn


def _fused_body(adj_ref, x_ref, w_ref, b_ref, out_ref,
                adjb_s, d_s, ps_s, u_s, xs_s):
    s = pl.program_id(0)

    @pl.when(s < NS)
    def _():
        rows = pl.ds(s * BS, BS)
        af8 = adj_ref[...].astype(PROP_DTYPE)
        adjb_s[rows, :] = af8
        ones = jnp.ones((N, 128), PROP_DTYPE)
        deg = jnp.dot(af8, ones, preferred_element_type=jnp.float32)
        d = jax.lax.rsqrt(deg[:, :1])
        d_s[rows, :] = d
        for b in range(B):
            xb = x_ref[b, rows, :]
            xs_s[rows, b * CIN:(b + 1) * CIN] = xb
            u_s[rows, b * CIN:(b + 1) * CIN] = (xb * d).astype(PROP_DTYPE)

    for t in range(NS):
        @pl.when(s == t + 1)
        def _(t=t):
            hi = (t + 1) * BS
            ps_s[t * BS:hi, :] = jnp.dot(
                adjb_s[t * BS:hi, :hi], u_s[:hi, :],
                preferred_element_type=jnp.float32)
            if t > 0:
                ps_s[:t * BS, :] += jnp.dot(
                    adjb_s[:t * BS, t * BS:hi], u_s[t * BS:hi, :],
                    preferred_element_type=jnp.float32)

    @pl.when(s == NS)
    def _():
        p = ps_s[...] * d_s[...]
        ps_s[...] = p
        u_s[...] = ((xs_s[...] - p) * d_s[...]).astype(PROP_DTYPE)

    @pl.when(s >= Q0)
    def _():
        rows = pl.ds((s - Q0) * BQ, BQ)
        acc = jnp.dot(adjb_s[rows, :], u_s[...],
                      preferred_element_type=jnp.float32)
        q = (acc * d_s[rows, :]).astype(jnp.bfloat16)
        p = ps_s[rows, :].astype(jnp.bfloat16)
        xb = xs_s[rows, :]
        w0 = w_ref[0, 0]
        w1 = w_ref[1, 0]
        w2 = w_ref[2, 0]
        wa = w0 + w1 + w2
        wb = (-(w1 + 2.0 * w2)).astype(jnp.bfloat16)
        wc = (-2.0 * w2).astype(jnp.bfloat16)
        bias = b_ref[0, 0, :]
        for b in range(B):
            sl = slice(b * CIN, (b + 1) * CIN)
            r = jnp.dot(xb[:, sl], wa, preferred_element_type=jnp.float32)
            r = r + jnp.dot(p[:, sl], wb, preferred_element_type=jnp.float32)
            r = r + jnp.dot(q[:, sl], wc, preferred_element_type=jnp.float32)
            out_ref[b] = jnp.maximum(r + bias, 0.0)


@jax.jit
def kernel(x, adj, weight, bias):
    return pl.pallas_call(
        _fused_body,
        grid=(Q0 + NQ,),
        in_specs=[
            pl.BlockSpec((BS, N), lambda s: (jnp.minimum(s, NS - 1), 0)),
            pl.BlockSpec((B, N, CIN), lambda s: (0, 0, 0)),
            pl.BlockSpec((3, 1, CIN, COUT), lambda s: (0, 0, 0, 0)),
            pl.BlockSpec((1, 1, COUT), lambda s: (0, 0, 0)),
        ],
        out_specs=pl.BlockSpec(
            (B, BQ, COUT), lambda s: (0, jnp.maximum(s - Q0, 0), 0)),
        out_shape=jax.ShapeDtypeStruct((B, N, COUT), jnp.float32),
        scratch_shapes=[
            pltpu.VMEM((N, N), PROP_DTYPE),
            pltpu.VMEM((N, 1), jnp.float32),
            pltpu.VMEM((N, B * CIN), jnp.float32),
            pltpu.VMEM((N, B * CIN), PROP_DTYPE),
            pltpu.VMEM((N, B * CIN), jnp.float32),
        ],
    )(adj, x, weight, bias)


# speedup vs baseline: 6.4711x; 1.0138x over previous
import jax
import jax.numpy as jnp
from jax.experimental import pallas as pl
from jax.experimental.pallas import tpu as pltpu

B, N, CIN, COUT = 2, 2048, 128, 128
BS = 512
PAIR = 2 * BS
NS = N // PAIR
BQ = 256
NQ = N // BQ
Q0 = NS + 1
PROP_DTYPE = jnp.float8_e4m3fn


def _fused_body(adj0_ref, adj1_ref, x_ref, w_ref, b_ref, out_ref,
                adjb_s, d_s, ps_s, u_s, xs_s):
    s = pl.program_id(0)

    @pl.when(s < NS)
    def _():
        ones = jnp.ones((N, 128), PROP_DTYPE)
        for h, aref in enumerate((adj0_ref, adj1_ref)):
            rows = pl.ds(s * PAIR + h * BS, BS)
            af8 = aref[...].astype(PROP_DTYPE)
            adjb_s[rows, :] = af8
            deg = jnp.dot(af8, ones, preferred_element_type=jnp.float32)
            d = jax.lax.rsqrt(deg[:, :1])
            d_s[rows, :] = d
            for b in range(B):
                xb = x_ref[b, rows, :]
                xs_s[rows, b * CIN:(b + 1) * CIN] = xb
                u_s[rows, b * CIN:(b + 1) * CIN] = (xb * d).astype(PROP_DTYPE)

    for t in range(NS):
        @pl.when(s == t + 1)
        def _(t=t):
            hi = (t + 1) * PAIR
            ps_s[t * PAIR:hi, :] = jnp.dot(
                adjb_s[t * PAIR:hi, :hi], u_s[:hi, :],
                preferred_element_type=jnp.float32)
            if t > 0:
                ps_s[:t * PAIR, :] += jnp.dot(
                    adjb_s[:t * PAIR, t * PAIR:hi], u_s[t * PAIR:hi, :],
                    preferred_element_type=jnp.float32)

    @pl.when(s == NS)
    def _():
        p = ps_s[...] * d_s[...]
        ps_s[...] = p
        u_s[...] = ((xs_s[...] - p) * d_s[...]).astype(PROP_DTYPE)

    @pl.when(s >= Q0)
    def _():
        rows = pl.ds((s - Q0) * BQ, BQ)
        acc = jnp.dot(adjb_s[rows, :], u_s[...],
                      preferred_element_type=jnp.float32)
        q = (acc * d_s[rows, :]).astype(jnp.bfloat16)
        p = ps_s[rows, :].astype(jnp.bfloat16)
        xb = xs_s[rows, :]
        w0 = w_ref[0, 0]
        w1 = w_ref[1, 0]
        w2 = w_ref[2, 0]
        wa = w0 + w1 + w2
        wb = (-(w1 + 2.0 * w2)).astype(jnp.bfloat16)
        wc = (-2.0 * w2).astype(jnp.bfloat16)
        bias = b_ref[0, 0, :]
        for b in range(B):
            sl = slice(b * CIN, (b + 1) * CIN)
            r = jnp.dot(xb[:, sl], wa, preferred_element_type=jnp.float32)
            r = r + jnp.dot(p[:, sl], wb, preferred_element_type=jnp.float32)
            r = r + jnp.dot(q[:, sl], wc, preferred_element_type=jnp.float32)
            out_ref[b] = jnp.maximum(r + bias, 0.0)


@jax.jit
def kernel(x, adj, weight, bias):
    return pl.pallas_call(
        _fused_body,
        grid=(Q0 + NQ,),
        in_specs=[
            pl.BlockSpec((BS, N), lambda s: (jnp.minimum(2 * s, 2 * NS - 2), 0)),
            pl.BlockSpec((BS, N), lambda s: (jnp.minimum(2 * s + 1, 2 * NS - 1), 0)),
            pl.BlockSpec((B, N, CIN), lambda s: (0, 0, 0)),
            pl.BlockSpec((3, 1, CIN, COUT), lambda s: (0, 0, 0, 0)),
            pl.BlockSpec((1, 1, COUT), lambda s: (0, 0, 0)),
        ],
        out_specs=pl.BlockSpec(
            (B, BQ, COUT), lambda s: (0, jnp.maximum(s - Q0, 0), 0)),
        out_shape=jax.ShapeDtypeStruct((B, N, COUT), jnp.float32),
        scratch_shapes=[
            pltpu.VMEM((N, N), PROP_DTYPE),
            pltpu.VMEM((N, 1), jnp.float32),
            pltpu.VMEM((N, B * CIN), jnp.float32),
            pltpu.VMEM((N, B * CIN), PROP_DTYPE),
            pltpu.VMEM((N, B * CIN), jnp.float32),
        ],
    )(adj, adj, x, weight, bias)


# speedup vs baseline: 6.8781x; 1.0629x over previous
import jax
import jax.numpy as jnp
from jax.experimental import pallas as pl
from jax.experimental.pallas import tpu as pltpu

B, N, CIN, COUT = 2, 2048, 128, 128
BS = 256
PAIR = 2 * BS
NS = N // PAIR
BQ = 512
NQ = N // BQ
Q0 = NS + 1
PROP_DTYPE = jnp.float8_e4m3fn


def _fused_body(adj0_ref, adj1_ref, x_ref, w_ref, b_ref, out_ref,
                adjb_s, d_s, ps_s, u_s):
    s = pl.program_id(0)

    @pl.when(s < NS)
    def _():
        ones = jnp.ones((N, 128), PROP_DTYPE)
        for h, aref in enumerate((adj0_ref, adj1_ref)):
            rows = pl.ds(s * PAIR + h * BS, BS)
            af8 = aref[...].astype(PROP_DTYPE)
            adjb_s[rows, :] = af8
            deg = jnp.dot(af8, ones, preferred_element_type=jnp.float32)
            d = jax.lax.rsqrt(deg[:, :1])
            d_s[rows, :] = d
            for b in range(B):
                u_s[rows, b * CIN:(b + 1) * CIN] = (
                    x_ref[b, rows, :] * d).astype(PROP_DTYPE)

    for t in range(NS):
        @pl.when(s == t + 1)
        def _(t=t):
            hi = (t + 1) * PAIR
            ps_s[t * PAIR:hi, :] = jnp.dot(
                adjb_s[t * PAIR:hi, :hi], u_s[:hi, :],
                preferred_element_type=jnp.float32)
            if t > 0:
                ps_s[:t * PAIR, :] += jnp.dot(
                    adjb_s[:t * PAIR, t * PAIR:hi], u_s[t * PAIR:hi, :],
                    preferred_element_type=jnp.float32)

    @pl.when(s == NS)
    def _():
        dd = d_s[...]
        for b in range(B):
            sl = slice(b * CIN, (b + 1) * CIN)
            u_s[:, sl] = ((x_ref[b] - ps_s[:, sl] * dd) * dd).astype(
                PROP_DTYPE)

    @pl.when(s >= Q0)
    def _():
        rows = pl.ds((s - Q0) * BQ, BQ)
        d = d_s[rows, :]
        acc = jnp.dot(adjb_s[rows, :], u_s[...],
                      preferred_element_type=jnp.float32)
        q = (acc * d).astype(jnp.bfloat16)
        p = (ps_s[rows, :] * d).astype(jnp.bfloat16)
        w0 = w_ref[0, 0]
        w1 = w_ref[1, 0]
        w2 = w_ref[2, 0]
        wa = w0 + w1 + w2
        wb = (-(w1 + 2.0 * w2)).astype(jnp.bfloat16)
        wc = (-2.0 * w2).astype(jnp.bfloat16)
        bias = b_ref[0, 0, :]
        for b in range(B):
            sl = slice(b * CIN, (b + 1) * CIN)
            r = jnp.dot(x_ref[b, rows, :], wa,
                        preferred_element_type=jnp.float32)
            r = r + jnp.dot(p[:, sl], wb, preferred_element_type=jnp.float32)
            r = r + jnp.dot(q[:, sl], wc, preferred_element_type=jnp.float32)
            out_ref[b] = jnp.maximum(r + bias, 0.0)


@jax.jit
def kernel(x, adj, weight, bias):
    return pl.pallas_call(
        _fused_body,
        grid=(Q0 + NQ,),
        in_specs=[
            pl.BlockSpec((BS, N), lambda s: (jnp.minimum(2 * s, 2 * NS - 2), 0)),
            pl.BlockSpec((BS, N), lambda s: (jnp.minimum(2 * s + 1, 2 * NS - 1), 0)),
            pl.BlockSpec((B, N, CIN), lambda s: (0, 0, 0)),
            pl.BlockSpec((3, 1, CIN, COUT), lambda s: (0, 0, 0, 0)),
            pl.BlockSpec((1, 1, COUT), lambda s: (0, 0, 0)),
        ],
        out_specs=pl.BlockSpec(
            (B, BQ, COUT), lambda s: (0, jnp.maximum(s - Q0, 0), 0)),
        out_shape=jax.ShapeDtypeStruct((B, N, COUT), jnp.float32),
        scratch_shapes=[
            pltpu.VMEM((N, N), PROP_DTYPE),
            pltpu.VMEM((N, 1), jnp.float32),
            pltpu.VMEM((N, B * CIN), jnp.float32),
            pltpu.VMEM((N, B * CIN), PROP_DTYPE),
        ],
    )(adj, adj, x, weight, bias)
